# baseline (speedup 1.0000x reference)
import functools

import jax
import jax.numpy as jnp
from jax import lax
from jax.experimental import pallas as pl
from jax.experimental.pallas import tpu as pltpu
from jax.experimental.pallas import tpu_sc as plsc

B = 1024
J = 26
C = 1000
ROW = J * C
NPTS = B * J
NW = 32
NWIN = ROW // 8
WPC = 5
NCHUNK = 22
WINS_W = WPC * NCHUNK
CHUNK_R = WPC * 8
KPAD = 16

_mesh = plsc.VectorSubcoreMesh(core_axis_name="c", subcore_axis_name="s")


@functools.partial(
    pl.kernel,
    mesh=_mesh,
    out_type=jax.ShapeDtypeStruct((ROW, B), jnp.int32),
    scratch_types=[
        pltpu.VMEM((NPTS + KPAD,), jnp.int32),
        pltpu.VMEM((NWIN + 22,), jnp.int32),
        pltpu.VMEM((CHUNK_R, B), jnp.int32),
        pltpu.VMEM((CHUNK_R, B), jnp.int32),
        pltpu.SemaphoreType.DMA,
        pltpu.SemaphoreType.DMA,
    ],
    compiler_params=pltpu.CompilerParams(needs_layout_passes=False),
)
def _onehot_sc(keys_hbm, bnd_hbm, zeros_hbm, out_hbm,
               keysv, bndv, buf0, buf1, s0, s1):
    bufs = (buf0, buf1)
    sems = (s0, s1)
    wid = lax.axis_index("s") * 2 + lax.axis_index("c")
    s_w = jnp.minimum(wid * NWIN // NW, NWIN - WINS_W)

    pltpu.sync_copy(keys_hbm, keysv.at[pl.ds(0, NPTS)])
    keysv[pl.ds(NPTS, KPAD)] = jnp.full((KPAD,), jnp.int32(1 << 30))
    pltpu.sync_copy(bnd_hbm, bndv)
    pltpu.sync_copy(zeros_hbm, buf0)
    pltpu.sync_copy(zeros_hbm, buf1)

    ones = jnp.full((16,), 1, jnp.int32)
    zeros_v = jnp.zeros((16,), jnp.int32)

    def scatter_chunk(buf, k, val):
        w0 = s_w + WPC * k
        bv = bndv[pl.ds(w0, 16)]
        start = bv[0]
        end = bv[WPC]
        c0 = w0 * 8
        p0 = c0 * B
        p1 = p0 + CHUNK_R * B
        base = start & jnp.int32(~15)
        nvec = (end - base + 15) >> 4

        def body(i, _):
            kv = keysv[pl.ds(base + i * 16, 16)]
            m = (kv >= p0) & (kv < p1)
            lr = (kv >> 10) - c0
            lc = kv & 1023
            plsc.store_scatter(buf, [lr, lc], val, mask=m)
            return 0

        lax.fori_loop(0, nvec, body, 0)

    handles = [None, None]
    for k in range(NCHUNK):
        slot = k % 2
        if handles[slot] is not None:
            handles[slot].wait()
            scatter_chunk(bufs[slot], k - 2, zeros_v)
        scatter_chunk(bufs[slot], k, ones)
        dst = out_hbm.at[pl.ds((s_w + WPC * k) * 8, CHUNK_R)]
        handles[slot] = pltpu.async_copy(bufs[slot], dst, sems[slot])
    handles[0].wait()
    handles[1].wait()


def kernel(x):
    xi = x.astype(jnp.int32)
    cpos = xi + (jnp.arange(J, dtype=jnp.int32) * C)[None, :]
    keys = cpos * B + jnp.arange(B, dtype=jnp.int32)[:, None]
    keys = jnp.sort(keys.reshape(-1))
    wstarts = jnp.arange(NWIN + 1, dtype=jnp.int32) * (8 * B)
    bnd = jnp.searchsorted(keys, wstarts).astype(jnp.int32)
    bnd = jnp.concatenate([bnd, jnp.full((21,), NPTS, jnp.int32)])
    zeros = jnp.zeros((CHUNK_R, B), jnp.int32)
    out_t = _onehot_sc(keys, bnd, zeros)
    return out_t.T


# speedup vs baseline: 3.9048x; 3.9048x over previous
import functools

import jax
import jax.numpy as jnp
from jax import lax
from jax.experimental import pallas as pl
from jax.experimental.pallas import tpu as pltpu
from jax.experimental.pallas import tpu_sc as plsc

B = 1024
J = 26
C = 1000
ROW = J * C
NPTS = B * J
NW = 32
NWIN = ROW // 8
WPC = 5
NCHUNK = 22
WINS_W = WPC * NCHUNK
CHUNK_R = WPC * 8
KPAD = 16
SENTINEL = 1 << 30

_mesh = plsc.VectorSubcoreMesh(core_axis_name="c", subcore_axis_name="s")


def _worker_starts():
    return jnp.minimum(jnp.arange(NW, dtype=jnp.int32) * NWIN // NW,
                       NWIN - WINS_W)


@functools.partial(
    pl.kernel,
    mesh=_mesh,
    out_type=jax.ShapeDtypeStruct((ROW, B), jnp.int32),
    scratch_types=[
        pltpu.VMEM((NPTS + KPAD,), jnp.int32),
        pltpu.VMEM((NW,), jnp.int32),
        pltpu.VMEM((CHUNK_R, B), jnp.int32),
        pltpu.VMEM((CHUNK_R, B), jnp.int32),
        pltpu.SemaphoreType.DMA,
        pltpu.SemaphoreType.DMA,
    ],
    compiler_params=pltpu.CompilerParams(needs_layout_passes=False),
)
def _onehot_sc(keys_hbm, starts_hbm, zeros_hbm, out_hbm,
               keysv, startsv, buf0, buf1, s0, s1):
    bufs = (buf0, buf1)
    sems = (s0, s1)
    wid = lax.axis_index("s") * 2 + lax.axis_index("c")
    s_w = jnp.minimum(wid * NWIN // NW, NWIN - WINS_W)

    pltpu.sync_copy(keys_hbm, keysv.at[pl.ds(0, NPTS)])
    keysv[pl.ds(NPTS, KPAD)] = jnp.full((KPAD,), jnp.int32(SENTINEL))
    pltpu.sync_copy(starts_hbm, startsv)
    pltpu.sync_copy(zeros_hbm, buf0)
    pltpu.sync_copy(zeros_hbm, buf1)

    ones = jnp.full((16,), 1, jnp.int32)
    zeros_v = jnp.zeros((16,), jnp.int32)

    half = startsv[pl.ds((wid >> 4) * 16, 16)]
    lane = lax.broadcasted_iota(jnp.int32, (16,), 0)
    start_pt = jnp.sum(jnp.where(lane == (wid & 15), half, 0))
    v0 = start_pt >> 4

    def walk_chunk(buf, vstart, k, val):
        c0 = (s_w + WPC * k) * 8
        p0 = c0 * B
        p1 = p0 + CHUNK_R * B

        def cond(v):
            kv = keysv[pl.ds(v * 16, 16)]
            return kv[0] < p1

        def body(v):
            kv = keysv[pl.ds(v * 16, 16)]
            m = (kv >= p0) & (kv < p1)
            lr = (kv >> 10) - c0
            lc = kv & 1023
            plsc.store_scatter(buf, [lr, lc], val, mask=m)
            return v + 1

        vend = lax.while_loop(cond, body, vstart)
        return jnp.maximum(vend - 1, vstart)

    handles = [None, None]
    saved = [None, None]
    vptr = v0
    for k in range(NCHUNK):
        slot = k % 2
        if handles[slot] is not None:
            handles[slot].wait()
            walk_chunk(bufs[slot], saved[slot], k - 2, zeros_v)
        saved[slot] = vptr
        vptr = walk_chunk(bufs[slot], vptr, k, ones)
        dst = out_hbm.at[pl.ds((s_w + WPC * k) * 8, CHUNK_R)]
        handles[slot] = pltpu.async_copy(bufs[slot], dst, sems[slot])
    handles[0].wait()
    handles[1].wait()


def kernel(x):
    xi = x.astype(jnp.int32)
    cpos = xi + (jnp.arange(J, dtype=jnp.int32) * C)[None, :]
    keys = cpos * B + jnp.arange(B, dtype=jnp.int32)[:, None]
    keys = jnp.sort(keys.reshape(-1))
    wfirst = _worker_starts() * (8 * B)
    starts = jnp.sum((keys[:, None] < wfirst[None, :]).astype(jnp.int32),
                     axis=0)
    zeros = jnp.zeros((CHUNK_R, B), jnp.int32)
    out_t = _onehot_sc(keys, starts, zeros)
    return out_t.T


# speedup vs baseline: 4.1223x; 1.0557x over previous
import functools

import jax
import jax.numpy as jnp
from jax import lax
from jax.experimental import pallas as pl
from jax.experimental.pallas import tpu as pltpu
from jax.experimental.pallas import tpu_sc as plsc

B = 1024
J = 26
C = 1000
ROW = J * C
NPTS = B * J
NW = 32
NWIN = ROW // 8
WPC = 6
NCHUNK = 17
WINS_W = WPC * NCHUNK
CHUNK_R = WPC * 8
KPAD = 16
SENTINEL = 1 << 30

_mesh = plsc.VectorSubcoreMesh(core_axis_name="c", subcore_axis_name="s")


def _worker_starts():
    return jnp.minimum(jnp.arange(NW, dtype=jnp.int32) * NWIN // NW,
                       NWIN - WINS_W)


@functools.partial(
    pl.kernel,
    mesh=_mesh,
    out_type=jax.ShapeDtypeStruct((ROW, B), jnp.int32),
    scratch_types=[
        pltpu.VMEM((NPTS + KPAD,), jnp.int32),
        pltpu.VMEM((NW,), jnp.int32),
        pltpu.VMEM((CHUNK_R, B), jnp.int32),
        pltpu.VMEM((CHUNK_R, B), jnp.int32),
        pltpu.SemaphoreType.DMA,
        pltpu.SemaphoreType.DMA,
    ],
    compiler_params=pltpu.CompilerParams(needs_layout_passes=False),
)
def _onehot_sc(keys_hbm, starts_hbm, zeros_hbm, out_hbm,
               keysv, startsv, buf0, buf1, s0, s1):
    bufs = (buf0, buf1)
    sems = (s0, s1)
    wid = lax.axis_index("s") * 2 + lax.axis_index("c")
    s_w = jnp.minimum(wid * NWIN // NW, NWIN - WINS_W)

    pltpu.sync_copy(keys_hbm, keysv.at[pl.ds(0, NPTS)])
    keysv[pl.ds(NPTS, KPAD)] = jnp.full((KPAD,), jnp.int32(SENTINEL))
    pltpu.sync_copy(starts_hbm, startsv)
    pltpu.sync_copy(zeros_hbm, buf0)
    pltpu.sync_copy(zeros_hbm, buf1)

    ones = jnp.full((16,), 1, jnp.int32)
    zeros_v = jnp.zeros((16,), jnp.int32)

    half = startsv[pl.ds((wid >> 4) * 16, 16)]
    lane = lax.broadcasted_iota(jnp.int32, (16,), 0)
    start_pt = jnp.sum(jnp.where(lane == (wid & 15), half, 0))
    v0 = start_pt >> 4

    def walk_chunk(buf, vstart, k, val):
        c0 = (s_w + WPC * k) * 8
        p0 = c0 * B
        p1 = p0 + CHUNK_R * B

        def cond(v):
            kv = keysv[pl.ds(v * 16, 16)]
            return kv[0] < p1

        def body(v):
            kv = keysv[pl.ds(v * 16, 16)]
            m = (kv >= p0) & (kv < p1)
            lr = (kv >> 10) - c0
            lc = kv & 1023
            plsc.store_scatter(buf, [lr, lc], val, mask=m)
            return v + 1

        vend = lax.while_loop(cond, body, vstart)
        return jnp.maximum(vend - 1, vstart)

    handles = [None, None]
    saved = [None, None]
    vptr = v0
    for k in range(NCHUNK):
        slot = k % 2
        if handles[slot] is not None:
            handles[slot].wait()
            walk_chunk(bufs[slot], saved[slot], k - 2, zeros_v)
        saved[slot] = vptr
        vptr = walk_chunk(bufs[slot], vptr, k, ones)
        dst = out_hbm.at[pl.ds((s_w + WPC * k) * 8, CHUNK_R)]
        handles[slot] = pltpu.async_copy(bufs[slot], dst, sems[slot])
    handles[0].wait()
    handles[1].wait()


def kernel(x):
    xi = x.astype(jnp.int32)
    cpos = xi + (jnp.arange(J, dtype=jnp.int32) * C)[None, :]
    keys = cpos * B + jnp.arange(B, dtype=jnp.int32)[:, None]
    keys = jnp.sort(keys.reshape(-1))
    wfirst = _worker_starts() * (8 * B)
    starts = jnp.sum((keys[:, None] < wfirst[None, :]).astype(jnp.int32),
                     axis=0)
    zeros = jnp.zeros((CHUNK_R, B), jnp.int32)
    out_t = _onehot_sc(keys, starts, zeros)
    return out_t.T
